# baseline (speedup 1.0000x reference)
import functools

import jax
import jax.numpy as jnp
from jax import lax
from jax.experimental import pallas as pl
from jax.experimental.pallas import tpu as pltpu
from jax.experimental.pallas import tpu_sc as plsc

HOUSE_DIM = 64
TIME_DIM = 64
NC, NS = 2, 16
NW = NC * NS
CHUNK = 128
MLP_BLK = 2048


def _gather_body(house_ids_hbm, time_ids_hbm, house_table, time_table,
                 house_out, time_out, hidx_v, tidx_v, hrows_v, trows_v, sem):
    wid = lax.axis_index("s") * NC + lax.axis_index("c")
    bpw = hrows_v.shape[0]
    base = wid * bpw
    pltpu.sync_copy(house_ids_hbm.at[pl.ds(base, bpw)], hidx_v)
    pltpu.sync_copy(time_ids_hbm.at[pl.ds(base, bpw)], tidx_v)
    copies = []
    for j in range(bpw // CHUNK):
        sl = pl.ds(j * CHUNK, CHUNK)
        copies.append(pltpu.async_copy(
            house_table.at[hidx_v.at[sl]], hrows_v.at[sl, :], sem))
        copies.append(pltpu.async_copy(
            time_table.at[tidx_v.at[sl]], trows_v.at[sl, :], sem))
    for c in copies:
        c.wait()
    pltpu.sync_copy(hrows_v, house_out.at[pl.ds(base, bpw)])
    pltpu.sync_copy(trows_v, time_out.at[pl.ds(base, bpw)])


def _sc_gather(house_ids, time_ids, house_table, time_table):
    B = house_ids.shape[0]
    bpw = B // NW
    return pl.kernel(
        _gather_body,
        out_type=(
            jax.ShapeDtypeStruct((B, HOUSE_DIM), jnp.float32),
            jax.ShapeDtypeStruct((B, TIME_DIM), jnp.float32),
        ),
        mesh=plsc.VectorSubcoreMesh(
            core_axis_name="c", subcore_axis_name="s",
            num_cores=NC, num_subcores=NS),
        scratch_types=[
            pltpu.VMEM((bpw,), jnp.int32),
            pltpu.VMEM((bpw,), jnp.int32),
            pltpu.VMEM((bpw, HOUSE_DIM), jnp.float32),
            pltpu.VMEM((bpw, TIME_DIM), jnp.float32),
            pltpu.SemaphoreType.DMA,
        ],
        compiler_params=pltpu.CompilerParams(use_tc_tiling_on_sc=False),
    )(house_ids, time_ids, house_table, time_table)


def _mlp_body(he_ref, te_ref, w1_ref, b1_ref, w2_ref, b2_ref, w3_ref, b3_ref,
              out_ref):
    h = jnp.dot(he_ref[...], w1_ref[:HOUSE_DIM, :],
                preferred_element_type=jnp.float32)
    h += jnp.dot(te_ref[...], w1_ref[HOUSE_DIM:, :],
                 preferred_element_type=jnp.float32)
    h = jnp.maximum(h + b1_ref[...], 0.0)
    h = jnp.maximum(
        jnp.dot(h, w2_ref[...], preferred_element_type=jnp.float32)
        + b2_ref[...], 0.0)
    out_ref[...] = (jnp.dot(h, w3_ref[...], preferred_element_type=jnp.float32)
                    + b3_ref[...])


def _tc_mlp(he, te, W1, b1, W2, b2, W3, b3):
    B = he.shape[0]
    d1 = W1.shape[1]
    d2 = W2.shape[1]
    grid = (B // MLP_BLK,)
    full = lambda shape: pl.BlockSpec(shape, lambda i: (0, 0))
    return pl.pallas_call(
        _mlp_body,
        grid=grid,
        in_specs=[
            pl.BlockSpec((MLP_BLK, HOUSE_DIM), lambda i: (i, 0)),
            pl.BlockSpec((MLP_BLK, TIME_DIM), lambda i: (i, 0)),
            full(W1.shape),
            full((1, d1)),
            full(W2.shape),
            full((1, d2)),
            full(W3.shape),
            full((1, 1)),
        ],
        out_specs=pl.BlockSpec((MLP_BLK, 1), lambda i: (i, 0)),
        out_shape=jax.ShapeDtypeStruct((B, 1), jnp.float32),
    )(he, te, W1, b1.reshape(1, d1), W2, b2.reshape(1, d2), W3,
      b3.reshape(1, 1))


def kernel(house_ids, time_ids, house_table, time_table, W1, b1, W2, b2, W3,
           b3):
    he, te = _sc_gather(house_ids.astype(jnp.int32),
                        time_ids.astype(jnp.int32),
                        house_table, time_table)
    return _tc_mlp(he, te, W1, b1, W2, b2, W3, b3)
